# baseline (speedup 1.0000x reference)
import jax
import jax.numpy as jnp
from jax import lax
from jax.experimental import pallas as pl
from jax.experimental.pallas import tpu as pltpu
from jax.experimental.pallas import tpu_sc as plsc

N = 10000
E = 320000
HID = 128

NC = 2
NS = 16
LANES = 16
NW = NC * NS
EB = 40
EPT = E // NW
KB1 = 10
NCH1 = EPT // (KB1 * EB)
NP = 10240
EBLK = 512

_MESH = plsc.VectorSubcoreMesh(
    core_axis_name="c", subcore_axis_name="s", num_cores=NC, num_subcores=NS)

_f32 = jnp.float32
_i32 = jnp.int32


def _pe_body(pe_h, src4_h, dst4_h, pej_h, pei_h,
             sidx2, didx2, rows_j, rows_i):
  cid = lax.axis_index("c")
  sid = lax.axis_index("s")
  wid = sid * NC + cid

  def _chunk(j, carry):
    pltpu.sync_copy(src4_h.at[wid, j], sidx2)
    pltpu.sync_copy(dst4_h.at[wid, j], didx2)

    def _batch(i, c2):
      base = wid * EPT + (j * KB1 + i) * EB
      pltpu.sync_copy(pe_h.at[sidx2.at[i]], rows_j)
      pltpu.sync_copy(rows_j, pej_h.at[pl.ds(base, EB)])
      pltpu.sync_copy(pe_h.at[didx2.at[i]], rows_i)
      pltpu.sync_copy(rows_i, pei_h.at[pl.ds(base, EB)])
      return c2
    lax.fori_loop(0, KB1, _batch, 0)
    return carry
  lax.fori_loop(0, NCH1, _chunk, 0)


_pe_gather = pl.kernel(
    _pe_body,
    out_type=[
        jax.ShapeDtypeStruct((E, HID), _f32),
        jax.ShapeDtypeStruct((E, HID), _f32),
    ],
    mesh=_MESH,
    scratch_types=[
        pltpu.VMEM((KB1, EB), _i32),
        pltpu.VMEM((KB1, EB), _i32),
        pltpu.VMEM((EB, HID), _f32),
        pltpu.VMEM((EB, HID), _f32),
    ],
)


NHALF = NP // NC
TRASH = NHALF
SROWS = NHALF + 8
NROW2 = NHALF // NS
EPT2 = E // NS
KB2 = 20
NCH2 = EPT2 // (KB2 * EB)


def _agg_body(xa_h, eb_h, g16_h, src5_h, dst6_h, agg_h, cnt_h,
              sidx2, didx2, g16b, xab, ebb, aggsh):
  cid = lax.axis_index("c")
  sid = lax.axis_index("s")
  lo = cid * NHALF

  def _fill(buf, val):
    def _row(i, carry):
      for c in range(HID // LANES):
        buf[i, pl.ds(c * LANES, LANES)] = jnp.full((LANES,), val, _f32)
      return carry
    lax.fori_loop(0, EB, _row, 0)

  def _clear():
    for r in range(NROW2 // EB):
      pltpu.sync_copy(xab, aggsh.at[pl.ds(sid * NROW2 + r * EB, EB)])

    @pl.when(sid == 0)
    def _():
      pltpu.sync_copy(xab.at[pl.ds(0, 8)], aggsh.at[pl.ds(TRASH, 8)])

  _fill(xab, 0.0)
  _clear()
  plsc.subcore_barrier()

  def _chunk(j, carry):
    pltpu.sync_copy(src5_h.at[sid, j], sidx2)
    pltpu.sync_copy(dst6_h.at[cid, sid, j], didx2)

    def _batch(i, c2):
      base = sid * EPT2 + (j * KB2 + i) * EB
      pltpu.sync_copy(xa_h.at[sidx2.at[i]], xab)
      pltpu.sync_copy(eb_h.at[pl.ds(base, EB)], ebb)
      pltpu.sync_copy(g16_h.at[pl.ds(base, EB)], g16b)

      def _edge(e, c3):
        gv = g16b[e, :]
        for c in range(HID // LANES):
          sl = pl.ds(c * LANES, LANES)
          ebb[e, sl] = jnp.maximum(xab[e, sl] * gv + ebb[e, sl], 0.0)
        return c3
      lax.fori_loop(0, EB, _edge, 0)
      pltpu.sync_copy(ebb, aggsh.at[didx2.at[i]], add=True)
      return c2
    lax.fori_loop(0, KB2, _batch, 0)
    return carry
  lax.fori_loop(0, NCH2, _chunk, 0)
  plsc.subcore_barrier()

  pltpu.sync_copy(aggsh.at[pl.ds(sid * NROW2, NROW2)],
                  agg_h.at[pl.ds(lo + sid * NROW2, NROW2)])
  plsc.subcore_barrier()

  _fill(xab, 0.0)
  _clear()
  _fill(ebb, 1.0)
  plsc.subcore_barrier()

  def _cchunk(j, carry):
    pltpu.sync_copy(dst6_h.at[cid, sid, j], didx2)

    def _cbatch(i, c2):
      pltpu.sync_copy(ebb, aggsh.at[didx2.at[i]], add=True)
      return c2
    lax.fori_loop(0, KB2, _cbatch, 0)
    return carry
  lax.fori_loop(0, NCH2, _cchunk, 0)
  plsc.subcore_barrier()

  pltpu.sync_copy(aggsh.at[pl.ds(sid * NROW2, NROW2)],
                  cnt_h.at[pl.ds(lo + sid * NROW2, NROW2)])


_agg = pl.kernel(
    _agg_body,
    out_type=[
        jax.ShapeDtypeStruct((NP, HID), _f32),
        jax.ShapeDtypeStruct((NP, HID), _f32),
    ],
    mesh=_MESH,
    scratch_types=[
        pltpu.VMEM((KB2, EB), _i32),
        pltpu.VMEM((KB2, EB), _i32),
        pltpu.VMEM((EB, LANES), _f32),
        pltpu.VMEM((EB, HID), _f32),
        pltpu.VMEM((EB, HID), _f32),
        pltpu.VMEM_SHARED((SROWS, HID), _f32),
    ],
)


def _in_body(xin_ref, win_ref, a0_ref, d0_ref, x_ref, xa_ref):
  x = jnp.maximum(
      jnp.dot(xin_ref[...], win_ref[...], preferred_element_type=_f32), 0.0)
  x_ref[...] = x
  xa_ref[...] = jnp.dot(x, a0_ref[...],
                        preferred_element_type=_f32) + d0_ref[0:1, :]


def _edge_tc_body(rel_ref, pej_ref, pei_ref, cc_ref, w1a_ref, w1b_ref,
                  w2_ref, eb0_ref, eb1_ref, g0_ref, g1_ref):
  h = jnp.maximum(
      jnp.dot(pej_ref[...], w1a_ref[...], preferred_element_type=_f32)
      + jnp.dot(pei_ref[...], w1b_ref[...], preferred_element_type=_f32), 0.0)
  w2 = w2_ref[...]
  gl0 = jnp.sum(h[:, :64] * w2[0:1, :64], axis=1) + w2[2, 0]
  gl1 = jnp.sum(h[:, 64:] * w2[1:2, :64], axis=1) + w2[3, 0]
  g0 = jax.nn.sigmoid(gl0)
  g1 = jax.nn.sigmoid(gl1)
  eb = jnp.dot(rel_ref[...], cc_ref[...], preferred_element_type=_f32)
  eb0_ref[...] = eb[:, :HID] * g0[:, None]
  eb1_ref[...] = eb[:, HID:] * g1[:, None]
  g0_ref[...] = jnp.broadcast_to(g0[:, None], (EBLK, LANES))
  g1_ref[...] = jnp.broadcast_to(g1[:, None], (EBLK, LANES))


def _upd_mid_body(x_ref, p_ref, c_ref, u1_ref, u2_ref,
                  bu_ref, an_ref, dn_ref, x_out, xa_out):
  c = jnp.maximum(c_ref[...], 1.0)
  agg = p_ref[:N, :] / c
  xn = jnp.maximum(
      jnp.dot(x_ref[...], u1_ref[...], preferred_element_type=_f32)
      + jnp.dot(agg, u2_ref[...], preferred_element_type=_f32)
      + bu_ref[0:1, :], 0.0)
  x_out[...] = xn
  xa_out[...] = jnp.dot(xn, an_ref[...],
                        preferred_element_type=_f32) + dn_ref[0:1, :]


def _upd_out_body(x_ref, p_ref, c_ref, u1_ref, u2_ref,
                  bu_ref, wo_ref, bo_ref, out_ref):
  c = jnp.maximum(c_ref[...], 1.0)
  agg = p_ref[:N, :] / c
  xn = jnp.maximum(
      jnp.dot(x_ref[...], u1_ref[...], preferred_element_type=_f32)
      + jnp.dot(agg, u2_ref[...], preferred_element_type=_f32)
      + bu_ref[0:1, :], 0.0)
  out_ref[...] = jnp.dot(xn, wo_ref[...],
                         preferred_element_type=_f32) + bo_ref[0:1, :]


def _pad8(v):
  return jnp.pad(v.reshape(1, -1), ((0, 7), (0, 0)))


def kernel(entity_embs, pe, edge_index, relation_embs_per_edge,
           W_in, b_in, W_rel, b_rel, W_msg, b_msg, W_g1, b_g1, W_g2, b_g2,
           W_upd, b_upd, W_out, b_out):
  src = edge_index[0].astype(_i32)
  dst = edge_index[1].astype(_i32)
  src4 = src.reshape(NW, NCH1, KB1, EB)
  dst4 = dst.reshape(NW, NCH1, KB1, EB)
  src5 = src.reshape(NS, NCH2, KB2, EB)
  dm0 = jnp.where(dst < NHALF, dst, TRASH)
  dm1 = jnp.where(dst >= NHALF, dst - NHALF, TRASH)
  dst6 = jnp.stack([dm0, dm1]).reshape(NC, NS, NCH2, KB2, EB)

  pe_tab = jnp.concatenate(
      [pe, jnp.ones((N, 1), _f32), jnp.zeros((N, HID - 32), _f32)], axis=1)

  A0, B0 = W_msg[0][:HID], W_msg[0][HID:]
  A1, B1 = W_msg[1][:HID], W_msg[1][HID:]
  ccat = jnp.concatenate([W_rel @ B0, W_rel @ B1], axis=1)
  d0 = _pad8(b_rel @ B0 + b_msg[0])
  d1 = _pad8(b_rel @ B1 + b_msg[1])

  w1a = jnp.zeros((HID, 128), _f32)
  w1a = w1a.at[0:31, 0:64].set(W_g1[0][:31]).at[0:31, 64:128].set(W_g1[1][:31])
  w1a = w1a.at[31, 0:64].set(b_g1[0]).at[31, 64:128].set(b_g1[1])
  w1b = jnp.zeros((HID, 128), _f32)
  w1b = w1b.at[0:31, 0:64].set(W_g1[0][31:]).at[0:31, 64:128].set(W_g1[1][31:])
  w2 = jnp.zeros((8, 128), _f32)
  w2 = w2.at[0, 0:64].set(W_g2[0][:, 0]).at[1, 0:64].set(W_g2[1][:, 0])
  w2 = w2.at[2, 0].set(b_g2[0][0]).at[3, 0].set(b_g2[1][0])

  xin = jnp.concatenate([entity_embs, pe, jnp.ones((N, 1), _f32)], axis=1)
  winp = jnp.concatenate([W_in, b_in.reshape(1, HID)], axis=0)

  u10, u20, bu0 = W_upd[0][:HID], W_upd[0][HID:], _pad8(b_upd[0])
  u11, u21, bu1 = W_upd[1][:HID], W_upd[1][HID:], _pad8(b_upd[1])
  bo = _pad8(b_out)

  pej, pei = _pe_gather(pe_tab, src4, dst4)

  x0, xa0 = pl.pallas_call(
      _in_body,
      out_shape=[jax.ShapeDtypeStruct((N, HID), _f32),
                 jax.ShapeDtypeStruct((N, HID), _f32)],
  )(xin, winp, A0, d0)

  nblk = E // EBLK
  eb0, eb1, g0b, g1b = pl.pallas_call(
      _edge_tc_body,
      grid=(nblk,),
      in_specs=[
          pl.BlockSpec((EBLK, HID), lambda i: (i, 0)),
          pl.BlockSpec((EBLK, HID), lambda i: (i, 0)),
          pl.BlockSpec((EBLK, HID), lambda i: (i, 0)),
          pl.BlockSpec((HID, 2 * HID), lambda i: (0, 0)),
          pl.BlockSpec((HID, 128), lambda i: (0, 0)),
          pl.BlockSpec((HID, 128), lambda i: (0, 0)),
          pl.BlockSpec((8, 128), lambda i: (0, 0)),
      ],
      out_specs=[
          pl.BlockSpec((EBLK, HID), lambda i: (i, 0)),
          pl.BlockSpec((EBLK, HID), lambda i: (i, 0)),
          pl.BlockSpec((EBLK, LANES), lambda i: (i, 0)),
          pl.BlockSpec((EBLK, LANES), lambda i: (i, 0)),
      ],
      out_shape=[
          jax.ShapeDtypeStruct((E, HID), _f32),
          jax.ShapeDtypeStruct((E, HID), _f32),
          jax.ShapeDtypeStruct((E, LANES), _f32),
          jax.ShapeDtypeStruct((E, LANES), _f32),
      ],
  )(relation_embs_per_edge, pej, pei, ccat, w1a, w1b, w2)

  p0, cnt = _agg(xa0, eb0, g0b, src5, dst6)
  cv = cnt[:N, 0:1]
  x1, xa1 = pl.pallas_call(
      _upd_mid_body,
      out_shape=[jax.ShapeDtypeStruct((N, HID), _f32),
                 jax.ShapeDtypeStruct((N, HID), _f32)],
  )(x0, p0, cv, u10, u20, bu0, A1, d1)

  p1, _cnt1 = _agg(xa1, eb1, g1b, src5, dst6)
  out = pl.pallas_call(
      _upd_out_body,
      out_shape=jax.ShapeDtypeStruct((N, HID), _f32),
  )(x1, p1, cv, u11, u21, bu1, W_out, bo)
  return out


# speedup vs baseline: 1.4330x; 1.4330x over previous
import jax
import jax.numpy as jnp
from jax import lax
from jax.experimental import pallas as pl
from jax.experimental.pallas import tpu as pltpu
from jax.experimental.pallas import tpu_sc as plsc

N = 10000
E = 320000
HID = 128

NC = 2
NS = 16
LANES = 16
NW = NC * NS
EB = 32
EB1 = 40
EPT = E // NW
KB1 = 5
NCH1 = EPT // (KB1 * EB1)
NP = 10240
EBLK = 512

_MESH = plsc.VectorSubcoreMesh(
    core_axis_name="c", subcore_axis_name="s", num_cores=NC, num_subcores=NS)

_f32 = jnp.float32
_i32 = jnp.int32


def _pe_body(pe_h, src4_h, dst4_h, pej_h, pei_h,
             sidx2, didx2, rows_j, rows_i, sem_j, sem_i, sem_w):
  cid = lax.axis_index("c")
  sid = lax.axis_index("s")
  wid = sid * NC + cid

  def _chunk(j, carry):
    pltpu.sync_copy(src4_h.at[wid, j], sidx2)
    pltpu.sync_copy(dst4_h.at[wid, j], didx2)

    def _batch(i, c2):
      base = wid * EPT + (j * KB1 + i) * EB1
      dj = pltpu.async_copy(pe_h.at[sidx2.at[i]], rows_j, sem_j)
      di = pltpu.async_copy(pe_h.at[didx2.at[i]], rows_i, sem_i)
      dj.wait()
      wj = pltpu.async_copy(rows_j, pej_h.at[pl.ds(base, EB1)], sem_w)
      di.wait()
      wi = pltpu.async_copy(rows_i, pei_h.at[pl.ds(base, EB1)], sem_w)
      wj.wait()
      wi.wait()
      return c2
    lax.fori_loop(0, KB1, _batch, 0)
    return carry
  lax.fori_loop(0, NCH1, _chunk, 0)


_pe_gather = pl.kernel(
    _pe_body,
    out_type=[
        jax.ShapeDtypeStruct((E, HID), _f32),
        jax.ShapeDtypeStruct((E, HID), _f32),
    ],
    mesh=_MESH,
    scratch_types=[
        pltpu.VMEM((KB1, EB1), _i32),
        pltpu.VMEM((KB1, EB1), _i32),
        pltpu.VMEM((EB1, HID), _f32),
        pltpu.VMEM((EB1, HID), _f32),
        pltpu.SemaphoreType.DMA,
        pltpu.SemaphoreType.DMA,
        pltpu.SemaphoreType.DMA,
    ],
)


NHALF = NP // NC
TRASH = NHALF
SROWS = NHALF + 8
NROW2 = NHALF // NS
EPT2 = E // NS
KB2 = 25
NCH2 = EPT2 // (KB2 * EB)


def _agg_body(xa_h, eb_h, g16_h, src5_h, dst6_h, agg_h, cnt_h,
              sidx2, didx2, g16b, xab, ebb, aggsh,
              sem_g, sem_e, sem_f, sem_s):
  cid = lax.axis_index("c")
  sid = lax.axis_index("s")
  lo = cid * NHALF

  def _fill(buf, val):
    def _row(i, carry):
      for c in range(HID // LANES):
        buf[i, pl.ds(c * LANES, LANES)] = jnp.full((LANES,), val, _f32)
      return carry
    lax.fori_loop(0, EB, _row, 0)

  def _clear():
    for r in range(NROW2 // EB):
      pltpu.sync_copy(xab.at[0], aggsh.at[pl.ds(sid * NROW2 + r * EB, EB)])

    @pl.when(sid == 0)
    def _():
      pltpu.sync_copy(xab.at[0, pl.ds(0, 8)], aggsh.at[pl.ds(TRASH, 8)])

  _fill(xab.at[0], 0.0)
  _clear()
  plsc.subcore_barrier()

  def _p1_chunk(j, carry):
    pltpu.sync_copy(src5_h.at[sid, j], sidx2)
    pltpu.sync_copy(dst6_h.at[cid, sid, j], didx2)

    def _issue_in(k):
      base = sid * EPT2 + (j * KB2 + k) * EB
      b = k % 2
      return (pltpu.async_copy(xa_h.at[sidx2.at[k]], xab.at[b], sem_g),
              pltpu.async_copy(eb_h.at[pl.ds(base, EB)], ebb.at[b], sem_e),
              pltpu.async_copy(g16_h.at[pl.ds(base, EB)], g16b.at[b], sem_f))

    def _compute(k):
      b = k % 2

      def _edge(e, c3):
        gv = g16b[b, e, :]
        for c in range(HID // LANES):
          sl = pl.ds(c * LANES, LANES)
          ebb[b, e, sl] = jnp.maximum(
              xab[b, e, sl] * gv + ebb[b, e, sl], 0.0)
        return c3
      lax.fori_loop(0, EB, _edge, 0)

    ins = _issue_in(0)
    scat = [None, None]
    for k in range(KB2):
      b = k % 2
      for d in ins:
        d.wait()
      if scat[b] is not None:
        scat[b].wait()
        scat[b] = None
      _compute(k)
      if k + 1 < KB2:
        nb = (k + 1) % 2
        if scat[nb] is not None:
          scat[nb].wait()
          scat[nb] = None
        ins = _issue_in(k + 1)
      scat[b] = pltpu.async_copy(
          ebb.at[b], aggsh.at[didx2.at[k]], sem_s, add=True)
    for d in scat:
      if d is not None:
        d.wait()
    return carry
  lax.fori_loop(0, NCH2, _p1_chunk, 0)
  plsc.subcore_barrier()

  pltpu.sync_copy(aggsh.at[pl.ds(sid * NROW2, NROW2)],
                  agg_h.at[pl.ds(lo + sid * NROW2, NROW2)])
  plsc.subcore_barrier()

  _fill(xab.at[0], 0.0)
  _clear()
  _fill(ebb.at[0], 1.0)
  plsc.subcore_barrier()

  def _p2_chunk(j, carry):
    pltpu.sync_copy(dst6_h.at[cid, sid, j], didx2)
    descs = [pltpu.async_copy(ebb.at[0], aggsh.at[didx2.at[k]], sem_s,
                              add=True)
             for k in range(KB2)]
    for d in descs:
      d.wait()
    return carry
  lax.fori_loop(0, NCH2, _p2_chunk, 0)
  plsc.subcore_barrier()

  pltpu.sync_copy(aggsh.at[pl.ds(sid * NROW2, NROW2)],
                  cnt_h.at[pl.ds(lo + sid * NROW2, NROW2)])


_agg = pl.kernel(
    _agg_body,
    out_type=[
        jax.ShapeDtypeStruct((NP, HID), _f32),
        jax.ShapeDtypeStruct((NP, HID), _f32),
    ],
    mesh=_MESH,
    scratch_types=[
        pltpu.VMEM((KB2, EB), _i32),
        pltpu.VMEM((KB2, EB), _i32),
        pltpu.VMEM((2, EB, LANES), _f32),
        pltpu.VMEM((2, EB, HID), _f32),
        pltpu.VMEM((2, EB, HID), _f32),
        pltpu.VMEM_SHARED((SROWS, HID), _f32),
        pltpu.SemaphoreType.DMA,
        pltpu.SemaphoreType.DMA,
        pltpu.SemaphoreType.DMA,
        pltpu.SemaphoreType.DMA,
    ],
)


def _in_body(xin_ref, win_ref, a0_ref, d0_ref, x_ref, xa_ref):
  x = jnp.maximum(
      jnp.dot(xin_ref[...], win_ref[...], preferred_element_type=_f32), 0.0)
  x_ref[...] = x
  xa_ref[...] = jnp.dot(x, a0_ref[...],
                        preferred_element_type=_f32) + d0_ref[0:1, :]


def _edge_tc_body(rel_ref, pej_ref, pei_ref, cc_ref, w1a_ref, w1b_ref,
                  w2_ref, eb0_ref, eb1_ref, g0_ref, g1_ref):
  h = jnp.maximum(
      jnp.dot(pej_ref[...], w1a_ref[...], preferred_element_type=_f32)
      + jnp.dot(pei_ref[...], w1b_ref[...], preferred_element_type=_f32), 0.0)
  w2 = w2_ref[...]
  gl0 = jnp.sum(h[:, :64] * w2[0:1, :64], axis=1) + w2[2, 0]
  gl1 = jnp.sum(h[:, 64:] * w2[1:2, :64], axis=1) + w2[3, 0]
  g0 = jax.nn.sigmoid(gl0)
  g1 = jax.nn.sigmoid(gl1)
  eb = jnp.dot(rel_ref[...], cc_ref[...], preferred_element_type=_f32)
  eb0_ref[...] = eb[:, :HID] * g0[:, None]
  eb1_ref[...] = eb[:, HID:] * g1[:, None]
  g0_ref[...] = jnp.broadcast_to(g0[:, None], (EBLK, LANES))
  g1_ref[...] = jnp.broadcast_to(g1[:, None], (EBLK, LANES))


def _upd_mid_body(x_ref, p_ref, c_ref, u1_ref, u2_ref,
                  bu_ref, an_ref, dn_ref, x_out, xa_out):
  c = jnp.maximum(c_ref[...], 1.0)
  agg = p_ref[:N, :] / c
  xn = jnp.maximum(
      jnp.dot(x_ref[...], u1_ref[...], preferred_element_type=_f32)
      + jnp.dot(agg, u2_ref[...], preferred_element_type=_f32)
      + bu_ref[0:1, :], 0.0)
  x_out[...] = xn
  xa_out[...] = jnp.dot(xn, an_ref[...],
                        preferred_element_type=_f32) + dn_ref[0:1, :]


def _upd_out_body(x_ref, p_ref, c_ref, u1_ref, u2_ref,
                  bu_ref, wo_ref, bo_ref, out_ref):
  c = jnp.maximum(c_ref[...], 1.0)
  agg = p_ref[:N, :] / c
  xn = jnp.maximum(
      jnp.dot(x_ref[...], u1_ref[...], preferred_element_type=_f32)
      + jnp.dot(agg, u2_ref[...], preferred_element_type=_f32)
      + bu_ref[0:1, :], 0.0)
  out_ref[...] = jnp.dot(xn, wo_ref[...],
                         preferred_element_type=_f32) + bo_ref[0:1, :]


def _pad8(v):
  return jnp.pad(v.reshape(1, -1), ((0, 7), (0, 0)))


def kernel(entity_embs, pe, edge_index, relation_embs_per_edge,
           W_in, b_in, W_rel, b_rel, W_msg, b_msg, W_g1, b_g1, W_g2, b_g2,
           W_upd, b_upd, W_out, b_out):
  src = edge_index[0].astype(_i32)
  dst = edge_index[1].astype(_i32)
  src4 = src.reshape(NW, NCH1, KB1, EB1)
  dst4 = dst.reshape(NW, NCH1, KB1, EB1)
  src5 = src.reshape(NS, NCH2, KB2, EB)
  dm0 = jnp.where(dst < NHALF, dst, TRASH)
  dm1 = jnp.where(dst >= NHALF, dst - NHALF, TRASH)
  dst6 = jnp.stack([dm0, dm1]).reshape(NC, NS, NCH2, KB2, EB)

  pe_tab = jnp.concatenate(
      [pe, jnp.ones((N, 1), _f32), jnp.zeros((N, HID - 32), _f32)], axis=1)

  A0, B0 = W_msg[0][:HID], W_msg[0][HID:]
  A1, B1 = W_msg[1][:HID], W_msg[1][HID:]
  ccat = jnp.concatenate([W_rel @ B0, W_rel @ B1], axis=1)
  d0 = _pad8(b_rel @ B0 + b_msg[0])
  d1 = _pad8(b_rel @ B1 + b_msg[1])

  w1a = jnp.zeros((HID, 128), _f32)
  w1a = w1a.at[0:31, 0:64].set(W_g1[0][:31]).at[0:31, 64:128].set(W_g1[1][:31])
  w1a = w1a.at[31, 0:64].set(b_g1[0]).at[31, 64:128].set(b_g1[1])
  w1b = jnp.zeros((HID, 128), _f32)
  w1b = w1b.at[0:31, 0:64].set(W_g1[0][31:]).at[0:31, 64:128].set(W_g1[1][31:])
  w2 = jnp.zeros((8, 128), _f32)
  w2 = w2.at[0, 0:64].set(W_g2[0][:, 0]).at[1, 0:64].set(W_g2[1][:, 0])
  w2 = w2.at[2, 0].set(b_g2[0][0]).at[3, 0].set(b_g2[1][0])

  xin = jnp.concatenate([entity_embs, pe, jnp.ones((N, 1), _f32)], axis=1)
  winp = jnp.concatenate([W_in, b_in.reshape(1, HID)], axis=0)

  u10, u20, bu0 = W_upd[0][:HID], W_upd[0][HID:], _pad8(b_upd[0])
  u11, u21, bu1 = W_upd[1][:HID], W_upd[1][HID:], _pad8(b_upd[1])
  bo = _pad8(b_out)

  pej, pei = _pe_gather(pe_tab, src4, dst4)

  x0, xa0 = pl.pallas_call(
      _in_body,
      out_shape=[jax.ShapeDtypeStruct((N, HID), _f32),
                 jax.ShapeDtypeStruct((N, HID), _f32)],
  )(xin, winp, A0, d0)

  nblk = E // EBLK
  eb0, eb1, g0b, g1b = pl.pallas_call(
      _edge_tc_body,
      grid=(nblk,),
      in_specs=[
          pl.BlockSpec((EBLK, HID), lambda i: (i, 0)),
          pl.BlockSpec((EBLK, HID), lambda i: (i, 0)),
          pl.BlockSpec((EBLK, HID), lambda i: (i, 0)),
          pl.BlockSpec((HID, 2 * HID), lambda i: (0, 0)),
          pl.BlockSpec((HID, 128), lambda i: (0, 0)),
          pl.BlockSpec((HID, 128), lambda i: (0, 0)),
          pl.BlockSpec((8, 128), lambda i: (0, 0)),
      ],
      out_specs=[
          pl.BlockSpec((EBLK, HID), lambda i: (i, 0)),
          pl.BlockSpec((EBLK, HID), lambda i: (i, 0)),
          pl.BlockSpec((EBLK, LANES), lambda i: (i, 0)),
          pl.BlockSpec((EBLK, LANES), lambda i: (i, 0)),
      ],
      out_shape=[
          jax.ShapeDtypeStruct((E, HID), _f32),
          jax.ShapeDtypeStruct((E, HID), _f32),
          jax.ShapeDtypeStruct((E, LANES), _f32),
          jax.ShapeDtypeStruct((E, LANES), _f32),
      ],
  )(relation_embs_per_edge, pej, pei, ccat, w1a, w1b, w2)

  p0, cnt = _agg(xa0, eb0, g0b, src5, dst6)
  cv = cnt[:N, 0:1]
  x1, xa1 = pl.pallas_call(
      _upd_mid_body,
      out_shape=[jax.ShapeDtypeStruct((N, HID), _f32),
                 jax.ShapeDtypeStruct((N, HID), _f32)],
  )(x0, p0, cv, u10, u20, bu0, A1, d1)

  p1, _cnt1 = _agg(xa1, eb1, g1b, src5, dst6)
  out = pl.pallas_call(
      _upd_out_body,
      out_shape=jax.ShapeDtypeStruct((N, HID), _f32),
  )(x1, p1, cv, u11, u21, bu1, W_out, bo)
  return out
